# baseline (speedup 1.0000x reference)
import jax
import jax.numpy as jnp
from jax import lax
from jax.experimental import pallas as pl
from jax.experimental.pallas import tpu as pltpu
from jax.experimental.pallas import tpu_sc as plsc

N = 10000
E = 320000
D_IN = 128
D_HID = 16
N_CLS = 2
N_GRAPHS = 64

NT = 32
PT = 320
NP = NT * PT
S = 4000
NC = E // S
G = 128
OWN_CAP = ((S + G - 1) // G) * G
NEG = -3.0e38


def _sc_body(x_hbm, dst_hbm, src_hbm, max_hbm, sum_hbm, deg_hbm,
             dstbuf, srcbuf, own_dst, own_src, gbuf,
             maxacc, sumacc, dega, gsem):
    cid = lax.axis_index("c")
    sid = lax.axis_index("s")
    wid = cid * 16 + sid
    lo = wid * PT
    iota16 = lax.iota(jnp.int32, 16)
    zeros16 = jnp.zeros((16,), jnp.float32)
    neg16 = jnp.full((16,), NEG, jnp.float32)
    e0vec = (iota16 == 0).astype(jnp.float32)

    def init_acc(i, _):
        maxacc[i // 8, pl.ds((i % 8) * 16, 16)] = neg16
        sumacc[i // 8, pl.ds((i % 8) * 16, 16)] = zeros16
        return 0
    lax.fori_loop(0, PT * 8, init_acc, 0)

    def init_deg(i, _):
        dega[pl.ds(i * 16, 16)] = zeros16
        return 0
    lax.fori_loop(0, PT // 16, init_deg, 0)

    def init_own(i, _):
        own_src[pl.ds(i * 16, 16)] = jnp.zeros((16,), jnp.int32)
        own_dst[pl.ds(i * 16, 16)] = jnp.zeros((16,), jnp.int32)
        return 0
    lax.fori_loop(0, OWN_CAP // 16, init_own, 0)

    def chunk_body(c, _):
        base_e = pl.multiple_of(c * S, 8)
        pltpu.sync_copy(dst_hbm.at[pl.ds(base_e, S)], dstbuf)
        pltpu.sync_copy(src_hbm.at[pl.ds(base_e, S)], srcbuf)

        def scan_g(i, cnt_vec):
            d16 = dstbuf[pl.ds(i * 16, 16)]
            s16 = srcbuf[pl.ds(i * 16, 16)]
            m = (d16 >= lo) & (d16 < lo + PT)
            pos = cnt_vec + plsc.cumsum(jnp.where(m, 1, 0)) - 1
            plsc.store_scatter(own_dst, [pos], d16, mask=m)
            plsc.store_scatter(own_src, [pos], s16, mask=m)
            return cnt_vec + plsc.all_reduce_population_count(m)

        cnt_vec = lax.fori_loop(0, S // 16, scan_g,
                                jnp.zeros((16,), jnp.int32))
        k = jnp.max(cnt_vec)

        def drain_g(g, _):
            goff = pl.multiple_of(g * G, G)
            idx = own_src.at[pl.ds(goff, G)]
            pltpu.async_copy(x_hbm.at[idx], gbuf, gsem).wait()
            ke = jnp.minimum(k - goff, G)

            def edge_j(j, _):
                grp = own_dst[pl.ds(goff + j - (j % 16), 16)]
                dnums = lax.GatherDimensionNumbers(
                    offset_dims=(), collapsed_slice_dims=(0,),
                    start_index_map=(0,))
                dvec = lax.gather(
                    grp, jnp.full((16, 1), j % 16, jnp.int32), dnums,
                    slice_sizes=(1,),
                    mode=lax.GatherScatterMode.PROMISE_IN_BOUNDS)
                dloc = dvec - lo
                for d in range(8):
                    col = d * 16 + iota16
                    gv = gbuf[j, pl.ds(d * 16, 16)]
                    a = plsc.load_gather(maxacc, [dloc, col])
                    plsc.store_scatter(maxacc, [dloc, col],
                                       jnp.maximum(a, gv))
                    s = plsc.load_gather(sumacc, [dloc, col])
                    plsc.store_scatter(sumacc, [dloc, col], s + gv)
                dg = plsc.load_gather(dega, [dloc])
                plsc.store_scatter(dega, [dloc], dg + 1.0,
                                   mask=(iota16 == 0))
                return 0

            lax.fori_loop(0, ke, edge_j, 0)
            return 0

        ng = (k + G - 1) // G
        lax.fori_loop(0, ng, drain_g, 0)
        return 0

    lax.fori_loop(0, NC, chunk_body, 0)

    wlo = pl.multiple_of(lo, 8)
    pltpu.sync_copy(maxacc, max_hbm.at[pl.ds(wlo, PT)])
    pltpu.sync_copy(sumacc, sum_hbm.at[pl.ds(wlo, PT)])
    pltpu.sync_copy(dega, deg_hbm.at[pl.ds(wlo, PT)])


def _sc_aggregate(x, dst, src):
    mesh = plsc.VectorSubcoreMesh(core_axis_name="c", subcore_axis_name="s")
    run = pl.kernel(
        _sc_body,
        out_type=(jax.ShapeDtypeStruct((NP, 128), jnp.float32),
                  jax.ShapeDtypeStruct((NP, 128), jnp.float32),
                  jax.ShapeDtypeStruct((NP,), jnp.float32)),
        mesh=mesh,
        scratch_types=[
            pltpu.VMEM((S,), jnp.int32),
            pltpu.VMEM((S,), jnp.int32),
            pltpu.VMEM((OWN_CAP,), jnp.int32),
            pltpu.VMEM((OWN_CAP,), jnp.int32),
            pltpu.VMEM((G, 128), jnp.float32),
            pltpu.VMEM((PT, 128), jnp.float32),
            pltpu.VMEM((PT, 128), jnp.float32),
            pltpu.VMEM((PT,), jnp.float32),
            pltpu.SemaphoreType.DMA,
        ],
        compiler_params=pltpu.CompilerParams(needs_layout_passes=False),
    )
    return run(x, dst, src)


def _tail_body(maxraw_ref, sumraw_ref, degraw_ref, x_ref, batch_ref,
               lin_l_w_ref, lin_l_b_ref, lin_r_w_ref, bn_g_ref, bn_b_ref,
               fc_w_ref, fc_b_ref, out_ref):
    maxraw = maxraw_ref[...][:N]
    sumraw = sumraw_ref[...][:N]
    x = x_ref[...]
    deg = degraw_ref[...][:N]
    w_mean_t = lin_l_w_ref[...][:, :D_IN].T
    w_max_t = lin_l_w_ref[...][:, D_IN:].T
    hmax = jnp.where(deg > 0.0, maxraw, 0.0)
    h = (jnp.dot(sumraw, w_mean_t, preferred_element_type=jnp.float32)
         / jnp.clip(deg, 1.0, None)
         + jnp.dot(hmax, w_max_t, preferred_element_type=jnp.float32)
         + lin_l_b_ref[...]
         + jnp.dot(x, lin_r_w_ref[...].T, preferred_element_type=jnp.float32))
    mu = jnp.mean(h, axis=0, keepdims=True)
    var = jnp.mean((h - mu) ** 2, axis=0, keepdims=True)
    h = (h - mu) / jnp.sqrt(var + 1e-5) * bn_g_ref[...] + bn_b_ref[...]
    h = jax.nn.relu(h)
    b = batch_ref[...]
    onehot = (b[None, :, 0] == jax.lax.broadcasted_iota(
        jnp.int32, (N_GRAPHS, N), 0)).astype(jnp.float32)
    counts = jnp.sum(onehot, axis=1, keepdims=True)
    pooled = jnp.dot(onehot, h, preferred_element_type=jnp.float32) \
        / jnp.clip(counts, 1.0, None)
    out_ref[...] = jnp.dot(pooled, fc_w_ref[...].T,
                           preferred_element_type=jnp.float32) + fc_b_ref[...]


def kernel(x, edge_index, edge_attr, batch, lin_l_w, lin_l_b, lin_r_w,
           bn_gamma, bn_beta, fc_w, fc_b):
    dst = edge_index[0]
    src = edge_index[1]

    maxraw, sumraw, degraw = _sc_aggregate(x, dst, src)

    out = pl.pallas_call(
        _tail_body,
        out_shape=jax.ShapeDtypeStruct((N_GRAPHS, N_CLS), jnp.float32),
    )(maxraw, sumraw, degraw[:, None], x, batch[:, None], lin_l_w,
      lin_l_b[None, :],
      lin_r_w, bn_gamma[None, :], bn_beta[None, :], fc_w, fc_b[None, :])
    return out


# speedup vs baseline: 1.0198x; 1.0198x over previous
import jax
import jax.numpy as jnp
from jax import lax
from jax.experimental import pallas as pl
from jax.experimental.pallas import tpu as pltpu
from jax.experimental.pallas import tpu_sc as plsc

N = 10000
E = 320000
D_IN = 128
D_HID = 16
N_CLS = 2
N_GRAPHS = 64

NT = 32
PT = 320
NP = NT * PT
SB = PT + 8
S = 4000
NC = E // S
G = 128
OWN_CAP = ((S + G - 1) // G) * G
NEG = -3.0e38


def _sc_body(x_hbm, dst_hbm, src_hbm, max_hbm, sum_hbm, deg_hbm,
             dstbuf, srcbuf, own_dst, own_src, gbuf,
             maxacc, dega, sumsp, gsem):
    cid = lax.axis_index("c")
    sid = lax.axis_index("s")
    wid = cid * 16 + sid
    lo = wid * PT
    sbase = sid * SB
    iota16 = lax.iota(jnp.int32, 16)
    zeros16 = jnp.zeros((16,), jnp.float32)
    neg16 = jnp.full((16,), NEG, jnp.float32)
    ones16 = jnp.ones((16,), jnp.float32)

    def init_acc(i, _):
        maxacc[i // 8, pl.ds((i % 8) * 16, 16)] = neg16
        return 0
    lax.fori_loop(0, PT * 8, init_acc, 0)

    def init_deg(i, _):
        dega[pl.ds(i * 16, 16)] = zeros16
        return 0
    lax.fori_loop(0, (SB + 8) // 16, init_deg, 0)

    def init_gbuf(i, _):
        gbuf[i // 8, pl.ds((i % 8) * 16, 16)] = zeros16
        return 0
    lax.fori_loop(0, G * 8, init_gbuf, 0)

    sb = pl.multiple_of(sbase, 8)
    pltpu.sync_copy(gbuf, sumsp.at[pl.ds(sb, G)])
    pltpu.sync_copy(gbuf, sumsp.at[pl.ds(sb + G, G)])
    pltpu.sync_copy(gbuf.at[pl.ds(0, SB - 2 * G)],
                    sumsp.at[pl.ds(sb + 2 * G, SB - 2 * G)])

    def init_own(i, _):
        own_src[pl.ds(i * 16, 16)] = jnp.zeros((16,), jnp.int32)
        own_dst[pl.ds(i * 16, 16)] = jnp.full((16,), sbase + PT, jnp.int32)
        return 0
    lax.fori_loop(0, OWN_CAP // 16, init_own, 0)

    def chunk_body(c, _):
        base_e = pl.multiple_of(c * S, 8)
        pltpu.sync_copy(dst_hbm.at[pl.ds(base_e, S)], dstbuf)
        pltpu.sync_copy(src_hbm.at[pl.ds(base_e, S)], srcbuf)

        def scan_g(i, cnt_vec):
            d16 = dstbuf[pl.ds(i * 16, 16)]
            s16 = srcbuf[pl.ds(i * 16, 16)]
            m = (d16 >= lo) & (d16 < lo + PT)
            pos = cnt_vec + plsc.cumsum(jnp.where(m, 1, 0)) - 1
            plsc.store_scatter(own_dst, [pos], d16 - (lo - sbase), mask=m)
            plsc.store_scatter(own_src, [pos], s16, mask=m)
            return cnt_vec + plsc.all_reduce_population_count(m)

        cnt_vec = lax.fori_loop(0, S // 16, scan_g,
                                jnp.zeros((16,), jnp.int32))
        k = jnp.max(cnt_vec)
        kG = ((k + G - 1) // G) * G

        trash = jnp.full((16,), sbase + PT, jnp.int32)

        def pad_g(i, _):
            off = pl.multiple_of(i * 16, 16)
            v = own_dst[pl.ds(off, 16)]
            own_dst[pl.ds(off, 16)] = jnp.where(off + iota16 < k, v, trash)
            return 0
        lax.fori_loop(k // 16, kG // 16, pad_g, 0)

        def drain_g(g, _):
            goff = pl.multiple_of(g * G, G)
            idx = own_src.at[pl.ds(goff, G)]
            pltpu.async_copy(x_hbm.at[idx], gbuf, gsem).wait()
            pltpu.sync_copy(gbuf, sumsp.at[own_dst.at[pl.ds(goff, G)]],
                            add=True)
            ke = jnp.minimum(k - goff, G)

            def grp_j(jg, _):
                agrp = own_dst[pl.ds(goff + jg * 16, 16)]
                dgrp = agrp - sbase
                plsc.addupdate_scatter(dega, [dgrp], ones16)
                for t in range(16):
                    d = dgrp[t]
                    row = jg * 16 + t
                    for dd in range(8):
                        sl = pl.ds(dd * 16, 16)
                        gv = gbuf[row, sl]
                        maxacc[d, sl] = jnp.maximum(maxacc[d, sl], gv)
                return 0

            lax.fori_loop(0, (ke + 15) // 16, grp_j, 0)
            return 0

        ng = (k + G - 1) // G
        lax.fori_loop(0, ng, drain_g, 0)
        return 0

    lax.fori_loop(0, NC, chunk_body, 0)

    wlo = pl.multiple_of(lo, 8)
    pltpu.sync_copy(maxacc.at[pl.ds(0, PT)], max_hbm.at[pl.ds(wlo, PT)])
    pltpu.sync_copy(sumsp.at[pl.ds(sb, PT)], sum_hbm.at[pl.ds(wlo, PT)])
    pltpu.sync_copy(dega.at[pl.ds(0, PT)], deg_hbm.at[pl.ds(wlo, PT)])


def _sc_aggregate(x, dst, src):
    mesh = plsc.VectorSubcoreMesh(core_axis_name="c", subcore_axis_name="s")
    run = pl.kernel(
        _sc_body,
        out_type=(jax.ShapeDtypeStruct((NP, 128), jnp.float32),
                  jax.ShapeDtypeStruct((NP, 128), jnp.float32),
                  jax.ShapeDtypeStruct((NP,), jnp.float32)),
        mesh=mesh,
        scratch_types=[
            pltpu.VMEM((S,), jnp.int32),
            pltpu.VMEM((S,), jnp.int32),
            pltpu.VMEM((OWN_CAP,), jnp.int32),
            pltpu.VMEM((OWN_CAP,), jnp.int32),
            pltpu.VMEM((G, 128), jnp.float32),
            pltpu.VMEM((PT + 8, 128), jnp.float32),
            pltpu.VMEM((SB + 8,), jnp.float32),
            pltpu.VMEM_SHARED((16 * SB, 128), jnp.float32),
            pltpu.SemaphoreType.DMA,
        ],
        compiler_params=pltpu.CompilerParams(needs_layout_passes=False),
    )
    return run(x, dst, src)


def _tail_body(maxraw_ref, sumraw_ref, degraw_ref, x_ref, batch_ref,
               lin_l_w_ref, lin_l_b_ref, lin_r_w_ref, bn_g_ref, bn_b_ref,
               fc_w_ref, fc_b_ref, out_ref):
    maxraw = maxraw_ref[...][:N]
    sumraw = sumraw_ref[...][:N]
    x = x_ref[...]
    deg = degraw_ref[...][:N]
    w_mean_t = lin_l_w_ref[...][:, :D_IN].T
    w_max_t = lin_l_w_ref[...][:, D_IN:].T
    hmax = jnp.where(deg > 0.0, maxraw, 0.0)
    h = (jnp.dot(sumraw, w_mean_t, preferred_element_type=jnp.float32)
         / jnp.clip(deg, 1.0, None)
         + jnp.dot(hmax, w_max_t, preferred_element_type=jnp.float32)
         + lin_l_b_ref[...]
         + jnp.dot(x, lin_r_w_ref[...].T, preferred_element_type=jnp.float32))
    mu = jnp.mean(h, axis=0, keepdims=True)
    var = jnp.mean((h - mu) ** 2, axis=0, keepdims=True)
    h = (h - mu) / jnp.sqrt(var + 1e-5) * bn_g_ref[...] + bn_b_ref[...]
    h = jax.nn.relu(h)
    b = batch_ref[...]
    onehot = (b[None, :, 0] == jax.lax.broadcasted_iota(
        jnp.int32, (N_GRAPHS, N), 0)).astype(jnp.float32)
    counts = jnp.sum(onehot, axis=1, keepdims=True)
    pooled = jnp.dot(onehot, h, preferred_element_type=jnp.float32) \
        / jnp.clip(counts, 1.0, None)
    out_ref[...] = jnp.dot(pooled, fc_w_ref[...].T,
                           preferred_element_type=jnp.float32) + fc_b_ref[...]


def kernel(x, edge_index, edge_attr, batch, lin_l_w, lin_l_b, lin_r_w,
           bn_gamma, bn_beta, fc_w, fc_b):
    dst = edge_index[0]
    src = edge_index[1]

    maxraw, sumraw, degraw = _sc_aggregate(x, dst, src)

    out = pl.pallas_call(
        _tail_body,
        out_shape=jax.ShapeDtypeStruct((N_GRAPHS, N_CLS), jnp.float32),
    )(maxraw, sumraw, degraw[:, None], x, batch[:, None], lin_l_w,
      lin_l_b[None, :],
      lin_r_w, bn_gamma[None, :], bn_beta[None, :], fc_w, fc_b[None, :])
    return out
